# Initial kernel scaffold; baseline (speedup 1.0000x reference)
import jax
import jax.numpy as jnp
from jax.experimental import pallas as pl


def kernel(x, edge_index, W1_rel, b1, W1_root, W2_rel, b2, W2_root):
    raise NotImplementedError("write your pallas kernel here")


# speedup vs baseline: 5.2293x; 5.2293x over previous
import functools

import jax
import jax.numpy as jnp
from jax import lax
from jax.experimental import pallas as pl
from jax.experimental.pallas import tpu as pltpu
from jax.experimental.pallas import tpu_sc as plsc

N_NODES = 10000
N_EDGES = 320000
D = 128

NC = 2
NS = 16
NW = NC * NS
EPW = N_EDGES // NW
CH = 80
NCH = EPW // CH
RPT = 624
TAIL = N_NODES - RPT * NS
ZR = 24

_mesh = plsc.VectorSubcoreMesh(core_axis_name="c", subcore_axis_name="s")


@functools.partial(
    pl.kernel,
    out_type=jax.ShapeDtypeStruct((NC, N_NODES, D), jnp.float32),
    mesh=_mesh,
    scratch_types=[
        pltpu.VMEM((CH,), jnp.int32),
        pltpu.VMEM((CH,), jnp.int32),
        pltpu.VMEM((CH, D), jnp.float32),
        pltpu.VMEM((ZR, D), jnp.float32),
        pltpu.VMEM_SHARED((N_NODES, D), jnp.float32),
        pltpu.SemaphoreType.DMA,
    ],
)
def _sc_agg(table, src, dst, out, idx_s, idx_d, rows, zbuf, acc, sem):
    cid = lax.axis_index("c")
    sid = lax.axis_index("s")
    wid = sid * NC + cid

    zv = jnp.zeros((16,), jnp.float32)
    for i in range(ZR):
        for j in range(D // 16):
            zbuf[i, pl.ds(j * 16, 16)] = zv

    def zero_body(i, carry):
        pltpu.sync_copy(zbuf, acc.at[pl.ds(sid * RPT + i * ZR, ZR)])
        return carry

    lax.fori_loop(0, RPT // ZR, zero_body, 0)

    @pl.when(sid == 0)
    def _zero_tail():
        pltpu.sync_copy(zbuf.at[pl.ds(0, TAIL)],
                        acc.at[pl.ds(RPT * NS, TAIL)])

    plsc.subcore_barrier()

    def edge_body(i, carry):
        base = wid * EPW + i * CH
        pltpu.sync_copy(src.at[pl.ds(base, CH)], idx_s)
        pltpu.sync_copy(dst.at[pl.ds(base, CH)], idx_d)
        pltpu.async_copy(table.at[idx_s], rows, sem).wait()
        pltpu.sync_copy(rows, acc.at[idx_d], add=True)
        return carry

    lax.fori_loop(0, NCH, edge_body, 0)
    plsc.subcore_barrier()

    pltpu.sync_copy(acc.at[pl.ds(sid * RPT, RPT)],
                    out.at[cid, pl.ds(sid * RPT, RPT)])

    @pl.when(sid == 0)
    def _copy_tail():
        pltpu.sync_copy(acc.at[pl.ds(RPT * NS, TAIL)],
                        out.at[cid, pl.ds(RPT * NS, TAIL)])


def _dense_body(p_ref, x_ref, wrel_ref, wroot_ref, b_ref, o_ref, *, relu):
    agg = p_ref[0] + p_ref[1]
    acc = jnp.dot(agg, wrel_ref[...], preferred_element_type=jnp.float32)
    acc = acc + jnp.dot(x_ref[...], wroot_ref[...],
                        preferred_element_type=jnp.float32)
    acc = acc + b_ref[...]
    o_ref[...] = jnp.maximum(acc, 0.0) if relu else acc


def _dense(partials, x, w_rel, w_root, b, relu):
    bn = 2000
    grid = (N_NODES // bn,)
    return pl.pallas_call(
        functools.partial(_dense_body, relu=relu),
        grid=grid,
        in_specs=[
            pl.BlockSpec((NC, bn, D), lambda i: (0, i, 0)),
            pl.BlockSpec((bn, D), lambda i: (i, 0)),
            pl.BlockSpec((D, D), lambda i: (0, 0)),
            pl.BlockSpec((D, D), lambda i: (0, 0)),
            pl.BlockSpec((1, D), lambda i: (0, 0)),
        ],
        out_specs=pl.BlockSpec((bn, D), lambda i: (i, 0)),
        out_shape=jax.ShapeDtypeStruct((N_NODES, D), jnp.float32),
    )(partials, x, w_rel, w_root, b.reshape(1, D))


def kernel(x, edge_index, W1_rel, b1, W1_root, W2_rel, b2, W2_root):
    ei = edge_index.astype(jnp.int32)
    src = ei[0]
    dst = ei[1]
    p1 = _sc_agg(x, src, dst)
    h = _dense(p1, x, W1_rel, W1_root, b1, relu=True)
    p2 = _sc_agg(h, src, dst)
    out = _dense(p2, h, W2_rel, W2_root, b2, relu=False)
    return out


# speedup vs baseline: 11.6289x; 2.2238x over previous
import functools

import jax
import jax.numpy as jnp
from jax import lax
from jax.experimental import pallas as pl
from jax.experimental.pallas import tpu as pltpu
from jax.experimental.pallas import tpu_sc as plsc

N_NODES = 10000
N_EDGES = 320000
D = 128

NC = 2
NS = 16
NW = NC * NS
EPW = N_EDGES // NW
CH = 40
NCH = EPW // CH
NB = 5
CSC = 50
SCH = NCH // CSC
NITER = CSC // NB
RPT = 624
TAIL = N_NODES - RPT * NS
ZR = 48

_mesh = plsc.VectorSubcoreMesh(core_axis_name="c", subcore_axis_name="s")


@functools.partial(
    pl.kernel,
    out_type=jax.ShapeDtypeStruct((NC, N_NODES, D), jnp.float32),
    mesh=_mesh,
    scratch_types=[
        pltpu.VMEM((CSC, CH), jnp.int32),
        pltpu.VMEM((CSC, CH), jnp.int32),
        pltpu.VMEM((NB, CH, D), jnp.float32),
        pltpu.VMEM((ZR, D), jnp.float32),
        pltpu.VMEM_SHARED((N_NODES, D), jnp.float32),
        pltpu.SemaphoreType.DMA((NB,)),
        pltpu.SemaphoreType.DMA((NB,)),
    ],
)
def _sc_agg(table, src, dst, out, sbuf, dbuf, rows, zbuf, acc, gsem, ssem):
    cid = lax.axis_index("c")
    sid = lax.axis_index("s")
    wid = sid * NC + cid

    zv = jnp.zeros((16,), jnp.float32)
    for i in range(ZR):
        for j in range(D // 16):
            zbuf[i, pl.ds(j * 16, 16)] = zv

    def zero_body(i, carry):
        pltpu.sync_copy(zbuf, acc.at[pl.ds(sid * RPT + i * ZR, ZR)])
        return carry

    lax.fori_loop(0, RPT // ZR, zero_body, 0)

    @pl.when(sid == 0)
    def _zero_tail():
        pltpu.sync_copy(zbuf.at[pl.ds(0, TAIL)],
                        acc.at[pl.ds(RPT * NS, TAIL)])

    plsc.subcore_barrier()

    def super_body(s, carry):
        pltpu.sync_copy(src.at[wid, s], sbuf)
        pltpu.sync_copy(dst.at[wid, s], dbuf)

        for b in range(NB):
            pltpu.async_copy(table.at[sbuf.at[b]], rows.at[b], gsem.at[b])

        def main(i, carry2):
            for b in range(NB):
                j = i * NB + b
                pltpu.make_async_copy(table.at[sbuf.at[j]], rows.at[b],
                                      gsem.at[b]).wait()
                pltpu.async_copy(rows.at[b], acc.at[dbuf.at[j]], ssem.at[b],
                                 add=True)
            for b in range(NB):
                j = i * NB + b
                pltpu.make_async_copy(rows.at[b], acc.at[dbuf.at[j]],
                                      ssem.at[b]).wait()
                pltpu.async_copy(table.at[sbuf.at[j + NB]], rows.at[b],
                                 gsem.at[b])
            return carry2

        lax.fori_loop(0, NITER - 1, main, 0)

        base = (NITER - 1) * NB
        for b in range(NB):
            pltpu.make_async_copy(table.at[sbuf.at[base + b]], rows.at[b],
                                  gsem.at[b]).wait()
            pltpu.async_copy(rows.at[b], acc.at[dbuf.at[base + b]],
                             ssem.at[b], add=True)
        for b in range(NB):
            pltpu.make_async_copy(rows.at[b], acc.at[dbuf.at[base + b]],
                                  ssem.at[b]).wait()
        return carry

    lax.fori_loop(0, SCH, super_body, 0)
    plsc.subcore_barrier()

    pltpu.sync_copy(acc.at[pl.ds(sid * RPT, RPT)],
                    out.at[cid, pl.ds(sid * RPT, RPT)])

    @pl.when(sid == 0)
    def _copy_tail():
        pltpu.sync_copy(acc.at[pl.ds(RPT * NS, TAIL)],
                        out.at[cid, pl.ds(RPT * NS, TAIL)])


def _dense_body(p_ref, x_ref, wrel_ref, wroot_ref, b_ref, o_ref, *, relu):
    agg = p_ref[0] + p_ref[1]
    acc = jnp.dot(agg, wrel_ref[...], preferred_element_type=jnp.float32)
    acc = acc + jnp.dot(x_ref[...], wroot_ref[...],
                        preferred_element_type=jnp.float32)
    acc = acc + b_ref[...]
    o_ref[...] = jnp.maximum(acc, 0.0) if relu else acc


def _dense(partials, x, w_rel, w_root, b, relu):
    bn = 2000
    grid = (N_NODES // bn,)
    return pl.pallas_call(
        functools.partial(_dense_body, relu=relu),
        grid=grid,
        in_specs=[
            pl.BlockSpec((NC, bn, D), lambda i: (0, i, 0)),
            pl.BlockSpec((bn, D), lambda i: (i, 0)),
            pl.BlockSpec((D, D), lambda i: (0, 0)),
            pl.BlockSpec((D, D), lambda i: (0, 0)),
            pl.BlockSpec((1, D), lambda i: (0, 0)),
        ],
        out_specs=pl.BlockSpec((bn, D), lambda i: (i, 0)),
        out_shape=jax.ShapeDtypeStruct((N_NODES, D), jnp.float32),
    )(partials, x, w_rel, w_root, b.reshape(1, D))


def kernel(x, edge_index, W1_rel, b1, W1_root, W2_rel, b2, W2_root):
    ei = edge_index.astype(jnp.int32)
    src = ei[0].reshape(NW, SCH, CSC, CH)
    dst = ei[1].reshape(NW, SCH, CSC, CH)
    p1 = _sc_agg(x, src, dst)
    h = _dense(p1, x, W1_rel, W1_root, b1, relu=True)
    p2 = _sc_agg(h, src, dst)
    out = _dense(p2, h, W2_rel, W2_root, b2, relu=False)
    return out
